# baseline (speedup 1.0000x reference)
import functools

import jax
import jax.numpy as jnp
from jax import lax
from jax.experimental import pallas as pl
from jax.experimental.pallas import tpu as pltpu
from jax.experimental.pallas import tpu_sc as plsc

_VOCAB = 100000
_D = 1024
_BATCH = 4
_SEQ = 2048
_NB = _BATCH * _SEQ

_info = plsc.get_sparse_core_info()
_NC = _info.num_cores
_NS = _info.num_subcores
_NW = _NC * _NS
_BPW = _NB // _NW
_C = 16
_NCHUNK = _BPW // _C
_NBUF = 4
_NSP = 2
_NGROUP = _NCHUNK // _NBUF

_mesh = plsc.VectorSubcoreMesh(core_axis_name="c", subcore_axis_name="s")


@functools.partial(
    pl.kernel,
    mesh=_mesh,
    out_type=jax.ShapeDtypeStruct((_NB, _D), jnp.float32),
    scratch_types=[
        pltpu.VMEM((_BPW,), jnp.int32),
        pltpu.VMEM((_NBUF, _C, _D), jnp.float32),
        pltpu.VMEM_SHARED((_NS, _NSP, _C, _D), jnp.float32),
        pltpu.SemaphoreType.DMA,
        pltpu.SemaphoreType.DMA,
        pltpu.SemaphoreType.DMA,
    ],
)
def _embed_sc(x_hbm, table_hbm, out_hbm, idx_v, buf_v, sp_v, gsem, psem, wsem):
    cid = lax.axis_index("c")
    sid = lax.axis_index("s")
    wid = sid * _NC + cid
    base = wid * _BPW

    def gather(j, b):
        pltpu.async_copy(
            table_hbm.at[idx_v.at[pl.ds(j * _C, _C)]], buf_v.at[b], gsem
        )

    def gather_wait(j, b):
        pltpu.make_async_copy(
            table_hbm.at[idx_v.at[pl.ds(j * _C, _C)]], buf_v.at[b], gsem
        ).wait()

    def push(b, s):
        pltpu.async_copy(buf_v.at[b], sp_v.at[sid, s], psem)

    def push_wait(b, s):
        pltpu.make_async_copy(buf_v.at[b], sp_v.at[sid, s], psem).wait()

    def write(j, s):
        pltpu.async_copy(
            sp_v.at[sid, s], out_hbm.at[pl.ds(base + j * _C, _C)], wsem
        )

    def write_wait(j, s):
        pltpu.make_async_copy(
            sp_v.at[sid, s], out_hbm.at[pl.ds(base + j * _C, _C)], wsem
        ).wait()

    pltpu.sync_copy(
        x_hbm.at[wid // (_SEQ // _BPW), pl.ds((wid % (_SEQ // _BPW)) * _BPW, _BPW)],
        idx_v,
    )

    for b in range(_NBUF):
        gather(b, b)

    for b in range(_NBUF):
        s = b % _NSP
        gather_wait(b, b)
        if b >= _NSP:
            write_wait(b - _NSP, s)
        push(b, s)
        push_wait(b, s)
        gather(b + _NBUF, b)
        write(b, s)

    def body(k, _):
        for b in range(_NBUF):
            j = _NBUF * k + b
            s = b % _NSP
            gather_wait(j, b)
            write_wait(j - _NSP, s)
            push(b, s)
            push_wait(b, s)
            gather(j + _NBUF, b)
            write(j, s)
        return 0

    lax.fori_loop(1, _NGROUP - 1, body, 0, unroll=False)

    for b in range(_NBUF):
        j = _NCHUNK - _NBUF + b
        s = b % _NSP
        gather_wait(j, b)
        write_wait(j - _NSP, s)
        push(b, s)
        push_wait(b, s)
        write(j, s)
    for b in range(_NBUF - _NSP, _NBUF):
        j = _NCHUNK - _NBUF + b
        write_wait(j, b % _NSP)


def kernel(x, table):
    out = _embed_sc(x, table)
    return out.reshape(_BATCH, _SEQ, _D)
